# baseline (speedup 1.0000x reference)
import functools

import jax
import jax.numpy as jnp
from jax import lax
from jax.experimental import pallas as pl
from jax.experimental.pallas import tpu as pltpu
from jax.experimental.pallas import tpu_sc as plsc

E = 320000
D = 128
G = 256
L = 16
NC = 2
NS = 16
NW = NC * NS
SPW = G // NW
C = 128
NBUF = 4
NV = D // L
CD = C * D
OFF_PAD = G + L

NEG_INF = float("-inf")


def _sc_body(x_hbm, off_hbm, w_hbm, b_hbm, out_hbm, buf, wv_ref, bv_ref,
             off_ref, stage, *sems):
    cid = lax.axis_index("c")
    sid = lax.axis_index("s")
    wid = sid * NC + cid
    g0 = wid * SPW

    pltpu.sync_copy(w_hbm, wv_ref)
    pltpu.sync_copy(b_hbm, bv_ref)
    pltpu.sync_copy(off_hbm, off_ref)

    wvec = [wv_ref[pl.ds(L * k, L)] for k in range(NV)]
    bv = bv_ref[...]
    iota = lax.iota(jnp.int32, L)

    def lane_shuffle(v, idx):
        return lax.gather(
            v, idx[:, None],
            lax.GatherDimensionNumbers(
                offset_dims=(), collapsed_slice_dims=(0,),
                start_index_map=(0,)),
            slice_sizes=(1,),
            mode=lax.GatherScatterMode.PROMISE_IN_BOUNDS)

    def off_at(idx):
        return off_ref[pl.ds(idx, L)][0]

    segb = [off_at(g0 + j) for j in range(SPW + 1)]
    e_lo, e_hi = segb[0], segb[SPW]
    nch = (e_hi - e_lo + (C - 1)) // C

    zero16 = jnp.zeros((L,), jnp.float32)
    ninf16 = jnp.full((L,), NEG_INF, jnp.float32)
    for j in range(SPW):
        for k in range(NV):
            stage[j, pl.ds(L * k, L)] = zero16
            stage[j, pl.ds(D + L * k, L)] = ninf16

    def issue(ci, slot):
        @pl.when(ci < nch)
        def _():
            start = e_lo + ci * C
            s_dma = jnp.minimum(start, E - C)
            pltpu.async_copy(
                x_hbm.at[pl.ds(s_dma * D, CD)],
                buf.at[pl.ds(slot * CD, CD)], sems[slot])

    def compute(ci, slot):
        lo = e_lo + ci * C
        s_dma = jnp.minimum(lo, E - C)
        j0 = lo - s_dma

        @pl.when(ci < nch)
        def _():
            pltpu.make_async_copy(
                x_hbm.at[pl.ds(0, CD)],
                buf.at[pl.ds(slot * CD, CD)], sems[slot]).wait()

        def seg_body(j, carry):
            sb = off_at(g0 + j)
            se = off_at(g0 + j + 1)
            a = jnp.maximum(sb, lo)
            bnd = jnp.minimum(se, lo + C)
            mj = bnd - a
            boff = slot * CD + (j0 + (a - lo)) * D

            @pl.when(mj > 0)
            def _():
                acc0 = (
                    tuple(stage[j, pl.ds(L * k, L)] for k in range(NV))
                    + tuple(stage[j, pl.ds(D + L * k, L)]
                            for k in range(NV)))

                def row_body(r, acc):
                    base = boff + r * D
                    xs = [buf[pl.ds(base + L * k, L)] for k in range(NV)]
                    p = xs[0] * wvec[0]
                    for k in range(1, NV):
                        p = p + xs[k] * wvec[k]
                    for sh in (8, 4, 2, 1):
                        p = p + lane_shuffle(p, iota ^ sh)
                    z = p + bv
                    ex = jnp.exp(-2.0 * jnp.abs(z))
                    wgt = jnp.sign(z) * (1.0 - ex) / (1.0 + ex)
                    news = tuple(acc[k] + xs[k] * wgt for k in range(NV))
                    newm = tuple(jnp.maximum(acc[NV + k], xs[k])
                                 for k in range(NV))
                    return news + newm

                acc = plsc.parallel_loop(
                    0, mj, unroll=4, carry=acc0)(row_body)
                for k in range(NV):
                    stage[j, pl.ds(L * k, L)] = acc[k]
                    stage[j, pl.ds(D + L * k, L)] = acc[NV + k]

            return carry

        lax.fori_loop(0, SPW, seg_body, 0)

    for b in range(NBUF - 1):
        issue(b, b)

    def group_body(t, carry):
        ci = NBUF * t
        for b in range(NBUF):
            issue(ci + b + (NBUF - 1), (b + NBUF - 1) % NBUF)
            compute(ci + b, b)
        return carry

    ngroups = (nch + (NBUF - 1)) // NBUF
    lax.fori_loop(0, ngroups, group_body, 0)

    pltpu.sync_copy(stage, out_hbm.at[pl.ds(g0, SPW), :])


@jax.jit
def kernel(edge_feats, segment_ids, W, b):
    ids32 = segment_ids.astype(jnp.int32)
    offsets = jnp.searchsorted(
        ids32, jnp.arange(G + 1, dtype=jnp.int32), side="left"
    ).astype(jnp.int32)
    off_pad = jnp.concatenate(
        [offsets, jnp.zeros((OFF_PAD - (G + 1),), jnp.int32)])
    x_flat = edge_feats.reshape(E * D)
    w_flat = W.reshape(D).astype(jnp.float32)
    b16 = jnp.broadcast_to(b.astype(jnp.float32), (L,))

    mesh = plsc.VectorSubcoreMesh(
        core_axis_name="c", subcore_axis_name="s",
        num_cores=NC, num_subcores=NS)
    f = pl.kernel(
        _sc_body,
        out_type=jax.ShapeDtypeStruct((G, 2 * D), jnp.float32),
        mesh=mesh,
        scratch_types=(
            [
                pltpu.VMEM((NBUF * C * D,), jnp.float32),
                pltpu.VMEM((D,), jnp.float32),
                pltpu.VMEM((L,), jnp.float32),
                pltpu.VMEM((OFF_PAD,), jnp.int32),
                pltpu.VMEM((SPW, 2 * D), jnp.float32),
            ]
            + [pltpu.SemaphoreType.DMA] * NBUF
        ),
    )
    return f(x_flat, off_pad, w_flat, b16)


# speedup vs baseline: 1.6600x; 1.6600x over previous
import functools

import jax
import jax.numpy as jnp
from jax import lax
from jax.experimental import pallas as pl
from jax.experimental.pallas import tpu as pltpu
from jax.experimental.pallas import tpu_sc as plsc

E = 320000
D = 128
G = 256
L = 16
NC = 2
NS = 16
NW = NC * NS
SPW = G // NW
C = 128
NBUF = 4
NV = D // L
CD = C * D
OFF_PAD = G + L

NEG_INF = float("-inf")


def _sc_body(x_hbm, ids_hbm, w_hbm, b_hbm, out_hbm, buf, wv_ref, bv_ref,
             off_ref, probe_ref, stage, psem, *sems):
    cid = lax.axis_index("c")
    sid = lax.axis_index("s")
    wid = sid * NC + cid
    g0 = wid * SPW

    pltpu.sync_copy(w_hbm, wv_ref)
    pltpu.sync_copy(b_hbm, bv_ref)

    wvec = [wv_ref[pl.ds(L * k, L)] for k in range(NV)]
    bv = bv_ref[...]
    iota = lax.iota(jnp.int32, L)

    def lane_shuffle(v, idx):
        return lax.gather(
            v, idx[:, None],
            lax.GatherDimensionNumbers(
                offset_dims=(), collapsed_slice_dims=(0,),
                start_index_map=(0,)),
            slice_sizes=(1,),
            mode=lax.GatherScatterMode.PROMISE_IN_BOUNDS)

    queries = g0 + iota
    lo = jnp.zeros((L,), jnp.int32)
    hi = jnp.full((L,), E, jnp.int32)

    def search_round(_, carry):
        lo, hi = carry
        mid = jnp.minimum((lo + hi) >> 1, E - 1)
        pltpu.async_copy(ids_hbm.at[mid], probe_ref, psem).wait()
        ge = probe_ref[...] >= queries
        active = lo < hi
        hi = jnp.where(active, jnp.where(ge, mid, hi), hi)
        lo = jnp.where(active, jnp.where(ge, lo, mid + 1), lo)
        return lo, hi

    lo, hi = lax.fori_loop(0, 19, search_round, (lo, hi))
    off_ref[pl.ds(0, L)] = hi
    off_ref[pl.ds(L, L)] = hi

    def off_at(j):
        return off_ref[pl.ds(j, L)][0]

    e_lo, e_hi = off_at(0), off_at(SPW)
    nch = (e_hi - e_lo + (C - 1)) // C

    zero16 = jnp.zeros((L,), jnp.float32)
    ninf16 = jnp.full((L,), NEG_INF, jnp.float32)
    for j in range(SPW):
        for k in range(NV):
            stage[j, pl.ds(L * k, L)] = zero16
            stage[j, pl.ds(D + L * k, L)] = ninf16

    def issue(ci, slot):
        @pl.when(ci < nch)
        def _():
            start = e_lo + ci * C
            s_dma = jnp.minimum(start, E - C)
            pltpu.async_copy(
                x_hbm.at[pl.ds(s_dma * D, CD)],
                buf.at[pl.ds(slot * CD, CD)], sems[slot])

    def compute(ci, slot):
        lo = e_lo + ci * C
        s_dma = jnp.minimum(lo, E - C)
        j0 = lo - s_dma

        @pl.when(ci < nch)
        def _():
            pltpu.make_async_copy(
                x_hbm.at[pl.ds(0, CD)],
                buf.at[pl.ds(slot * CD, CD)], sems[slot]).wait()

        def seg_body(j, carry):
            sb = off_at(j)
            se = off_at(j + 1)
            a = jnp.maximum(sb, lo)
            bnd = jnp.minimum(se, lo + C)
            mj = bnd - a
            boff = slot * CD + (j0 + (a - lo)) * D

            @pl.when(mj > 0)
            def _():
                acc0 = (
                    tuple(stage[j, pl.ds(L * k, L)] for k in range(NV))
                    + tuple(stage[j, pl.ds(D + L * k, L)]
                            for k in range(NV)))

                def row_body(r, acc):
                    base = boff + r * D
                    xs = [buf[pl.ds(base + L * k, L)] for k in range(NV)]
                    p = xs[0] * wvec[0]
                    for k in range(1, NV):
                        p = p + xs[k] * wvec[k]
                    for sh in (8, 4, 2, 1):
                        p = p + lane_shuffle(p, iota ^ sh)
                    z = p + bv
                    e2 = jnp.exp(z + z)
                    wgt = (e2 - 1.0) / (e2 + 1.0)
                    news = tuple(acc[k] + xs[k] * wgt for k in range(NV))
                    newm = tuple(jnp.maximum(acc[NV + k], xs[k])
                                 for k in range(NV))
                    return news + newm

                acc = plsc.parallel_loop(
                    0, mj, unroll=4, carry=acc0)(row_body)
                for k in range(NV):
                    stage[j, pl.ds(L * k, L)] = acc[k]
                    stage[j, pl.ds(D + L * k, L)] = acc[NV + k]

            return carry

        lax.fori_loop(0, SPW, seg_body, 0)

    for b in range(NBUF - 1):
        issue(b, b)

    def group_body(t, carry):
        ci = NBUF * t
        for b in range(NBUF):
            issue(ci + b + (NBUF - 1), (b + NBUF - 1) % NBUF)
            compute(ci + b, b)
        return carry

    ngroups = (nch + (NBUF - 1)) // NBUF
    lax.fori_loop(0, ngroups, group_body, 0)

    pltpu.sync_copy(stage, out_hbm.at[pl.ds(g0, SPW), :])


@jax.jit
def kernel(edge_feats, segment_ids, W, b):
    ids32 = segment_ids.astype(jnp.int32)
    x_flat = edge_feats.reshape(E * D)
    w_flat = W.reshape(D).astype(jnp.float32)
    b16 = jnp.broadcast_to(b.astype(jnp.float32), (L,))

    mesh = plsc.VectorSubcoreMesh(
        core_axis_name="c", subcore_axis_name="s",
        num_cores=NC, num_subcores=NS)
    f = pl.kernel(
        _sc_body,
        out_type=jax.ShapeDtypeStruct((G, 2 * D), jnp.float32),
        mesh=mesh,
        scratch_types=(
            [
                pltpu.VMEM((NBUF * C * D,), jnp.float32),
                pltpu.VMEM((D,), jnp.float32),
                pltpu.VMEM((L,), jnp.float32),
                pltpu.VMEM((2 * L,), jnp.int32),
                pltpu.VMEM((L,), jnp.int32),
                pltpu.VMEM((SPW, 2 * D), jnp.float32),
                pltpu.SemaphoreType.DMA,
            ]
            + [pltpu.SemaphoreType.DMA] * NBUF
        ),
    )
    return f(x_flat, ids32, w_flat, b16)
